# baseline (speedup 1.0000x reference)
import functools

import jax
import jax.numpy as jnp
from jax import lax
from jax.experimental import pallas as pl
from jax.experimental.pallas import tpu as pltpu
from jax.experimental.pallas import tpu_sc as plsc

_S = 200
_R = 4
_NBUF = 2


@functools.partial(jax.jit, static_argnames=("n_rows", "d"))
def _gather_rows(x2d, table, n_rows, d):
    info = plsc.get_sparse_core_info()
    nw = info.num_cores * info.num_subcores
    rows_w = n_rows // nw
    steps = rows_w // _R
    mesh = plsc.VectorSubcoreMesh(core_axis_name="c", subcore_axis_name="s")

    @functools.partial(
        pl.kernel,
        mesh=mesh,
        compiler_params=pltpu.CompilerParams(use_tc_tiling_on_sc=False),
        out_type=jax.ShapeDtypeStruct((n_rows, _S, d), jnp.float32),
        scratch_types=[
            pltpu.VMEM((_R, 104), jnp.int32),
            pltpu.VMEM((_R, 96), jnp.int32),
            pltpu.VMEM((_NBUF, _R, _S, d), jnp.float32),
            pltpu.VMEM_SHARED((10, d), jnp.float32),
            pltpu.SemaphoreType.DMA,
            pltpu.SemaphoreType.DMA,
            pltpu.SemaphoreType.DMA,
        ],
    )
    def k(x_hbm, t_hbm, out_hbm, raw_a, raw_b, rows_v, t_sh,
          sem_g, sem_w0, sem_w1):
        sid = lax.axis_index("s")
        wid = sid * info.num_cores + lax.axis_index("c")
        sem_w = (sem_w0, sem_w1)

        @pl.when(sid == 0)
        def _():
            pltpu.sync_copy(t_hbm, t_sh)

        plsc.subcore_barrier()

        def outer(io, carry):
            for b in range(_NBUF):
                i = _NBUF * io + b
                r0 = wid * rows_w + i * _R
                pltpu.sync_copy(x_hbm.at[pl.ds(r0, _R), pl.ds(0, 104)], raw_a)
                pltpu.sync_copy(x_hbm.at[pl.ds(r0, _R), pl.ds(104, 96)], raw_b)

                @pl.when(io >= 1)
                def _():
                    pltpu.make_async_copy(
                        rows_v.at[b], out_hbm.at[pl.ds(0, _R)], sem_w[b]
                    ).wait()

                copies = []
                for j in range(_R):
                    copies.append(pltpu.async_copy(
                        t_sh.at[raw_a.at[j]],
                        rows_v.at[b, j, pl.ds(0, 104)],
                        sem_g,
                    ))
                    copies.append(pltpu.async_copy(
                        t_sh.at[raw_b.at[j]],
                        rows_v.at[b, j, pl.ds(104, 96)],
                        sem_g,
                    ))
                for c in copies:
                    c.wait()
                pltpu.async_copy(
                    rows_v.at[b], out_hbm.at[pl.ds(r0, _R)], sem_w[b]
                )
            return carry

        lax.fori_loop(0, steps // _NBUF, outer, 0)
        for b in range(_NBUF):
            pltpu.make_async_copy(
                rows_v.at[b], out_hbm.at[pl.ds(0, _R)], sem_w[b]
            ).wait()

    return k(x2d, table)


def kernel(x, table):
    b, s = x.shape
    v, d = table.shape
    return _gather_rows(x, table, b, d)
